# baseline (speedup 1.0000x reference)
import functools

import jax
import jax.numpy as jnp
from jax import lax
from jax.experimental import pallas as pl
from jax.experimental.pallas import tpu as pltpu
from jax.experimental.pallas import tpu_sc as plsc

_K = 16
_BQ = 200
_BN = 1000
_NW = 32
_CH = 40
_F = 5


_NLANE = 128
_NG = 80
_R0 = 8


def _group_round(x3, cols3, npad):
    gmax = jnp.max(x3, axis=2)
    cand = jnp.where(x3 == gmax[:, :, None], cols3, npad)
    gmin = jnp.min(cand, axis=2)
    x3 = jnp.where(cand == gmin[:, :, None], -jnp.inf, x3)
    return x3, gmax, gmin


def _extract16(pv, pi, npad):
    outs = []
    for _ in range(_K):
        mx = jnp.max(pv, axis=1, keepdims=True)
        candi = jnp.where(pv == mx, pi, npad)
        imin = jnp.min(candi, axis=1, keepdims=True)
        outs.append(imin)
        pv = jnp.where((pv == mx) & (pi == imin), -jnp.inf, pv)
    return jnp.concatenate(outs, axis=1)


def _knn_body(xq_ref, xt_ref, idx_ref, pv_ref, pi_ref):
    xq = xq_ref[...]
    xt = xt_ref[...]
    npad = xt.shape[1]
    n = 10000
    qsq = jnp.sum(xq * xq, axis=1, keepdims=True)
    ksq = jnp.sum(xt * xt, axis=0, keepdims=True)
    mm = jnp.dot(xq, xt, preferred_element_type=jnp.float32)
    neg = -(qsq - 2.0 * mm + ksq)
    shape3 = (_BQ, _NG, _NLANE)
    cols3 = (_NLANE * lax.broadcasted_iota(jnp.int32, shape3, 1)
             + lax.broadcasted_iota(jnp.int32, shape3, 2))
    x3 = jnp.where(cols3 < n, neg.reshape(shape3), -jnp.inf)

    def round_body(r, x3):
        x3, gmax, gmin = _group_round(x3, cols3, npad)
        pv_ref[r] = gmax
        pi_ref[r] = gmin
        return x3

    lax.fori_loop(0, _R0, round_body, x3)
    pv = jnp.concatenate([pv_ref[r] for r in range(_R0)], axis=1)
    pi = jnp.concatenate([pi_ref[r] for r in range(_R0)], axis=1)
    idx_ref[...] = _extract16(pv, pi, npad)


def _knn(x):
    n, d = x.shape
    xt_pad = jnp.concatenate(
        [x.T, jnp.zeros((d, _NG * _NLANE - n), jnp.float32)], axis=1)
    return pl.pallas_call(
        _knn_body,
        grid=(n // _BQ,),
        in_specs=[
            pl.BlockSpec((_BQ, d), lambda i: (i, 0)),
            pl.BlockSpec((d, _NG * _NLANE), lambda i: (0, 0)),
        ],
        out_specs=pl.BlockSpec((_BQ, _K), lambda i: (i, 0)),
        out_shape=jax.ShapeDtypeStruct((n, _K), jnp.int32),
        scratch_shapes=[
            pltpu.VMEM((_R0, _BQ, _NG), jnp.float32),
            pltpu.VMEM((_R0, _BQ, _NG), jnp.int32),
        ],
    )(x, xt_pad)


def _ab_body(x_ref, w_ref, o_ref):
    o_ref[...] = jnp.dot(x_ref[...], w_ref[...],
                         preferred_element_type=jnp.float32)


def _ab(x, wcat):
    n = x.shape[0]
    return pl.pallas_call(
        _ab_body,
        out_shape=jax.ShapeDtypeStruct((n, wcat.shape[1]), jnp.float32),
    )(x, wcat)


def _sc_gather(table, idx3d):
    nch = idx3d.shape[1]
    rounds = nch // _F
    h = table.shape[1]
    mesh = plsc.VectorSubcoreMesh(core_axis_name="c", subcore_axis_name="s")

    @functools.partial(
        pl.kernel,
        mesh=mesh,
        out_type=jax.ShapeDtypeStruct((_NW * nch, _CH, h), jnp.float32),
        scratch_types=[
            pltpu.VMEM((nch, _CH), jnp.int32),
            pltpu.VMEM((_F, _CH, h), jnp.float32),
            pltpu.SemaphoreType.DMA,
        ],
    )
    def k(table_hbm, idx_hbm, out_hbm, idx_v, buf, sem):
        c = lax.axis_index("c")
        s = lax.axis_index("s")
        wid = s * 2 + c
        pltpu.sync_copy(idx_hbm.at[wid], idx_v)

        def round_body(r, carry):
            cps = [
                pltpu.async_copy(table_hbm.at[idx_v.at[r * _F + j]],
                                 buf.at[j], sem)
                for j in range(_F)
            ]
            for cp in cps:
                cp.wait()
            pltpu.sync_copy(buf, out_hbm.at[pl.ds(wid * nch + r * _F, _F)])
            return carry

        lax.fori_loop(0, rounds, round_body, 0)

    return k(table, idx3d)


def _msg_body(a_ref, b1_ref, bg3_ref, w2_ref, b2_ref, wg_ref, o_ref):
    a = a_ref[...] + b1_ref[...]
    hdim = a.shape[1]
    w2 = w2_ref[...]
    b2 = b2_ref[...]
    hacc = jnp.full(a.shape, -jnp.inf, jnp.float32)
    for k in range(_K):
        t = jnp.dot(jnp.maximum(a + bg3_ref[k][:, hdim:], 0.0), w2,
                    preferred_element_type=jnp.float32) + b2
        mx = jnp.max(t, axis=1, keepdims=True)
        lse = jnp.log(jnp.sum(jnp.exp(t - mx), axis=1, keepdims=True)) + mx
        hacc = jnp.maximum(hacc, t - lse)
    hw = jnp.dot(hacc, wg_ref[...],
                 preferred_element_type=jnp.float32) * (1.0 / 17.0)
    o_ref[...] = jnp.concatenate([hw, jnp.zeros_like(hw)], axis=1)


def _message(a, b1, bg3, w2, b2, wg):
    n, hdim = a.shape
    return pl.pallas_call(
        _msg_body,
        grid=(n // _BN,),
        in_specs=[
            pl.BlockSpec((_BN, hdim), lambda i: (i, 0)),
            pl.BlockSpec((1, hdim), lambda i: (0, 0)),
            pl.BlockSpec((_K, _BN, 2 * hdim), lambda i: (0, i, 0)),
            pl.BlockSpec((hdim, hdim), lambda i: (0, 0)),
            pl.BlockSpec((1, hdim), lambda i: (0, 0)),
            pl.BlockSpec((hdim, hdim), lambda i: (0, 0)),
        ],
        out_specs=pl.BlockSpec((_BN, 2 * hdim), lambda i: (i, 0)),
        out_shape=jax.ShapeDtypeStruct((n, 2 * hdim), jnp.float32),
    )(a, b1, bg3, w2, b2, wg)


def _final_body(hw_ref, hg3_ref, bgv_ref, wo1_ref, bo1_ref, wo2_ref,
                bo2_ref, o_ref):
    hdim = bgv_ref.shape[1]
    g = hw_ref[:, :hdim] + bgv_ref[...]
    for k in range(_K):
        g = g + hg3_ref[k][:, :hdim]
    t = jnp.dot(g, wo1_ref[...], preferred_element_type=jnp.float32)
    t = t + bo1_ref[...]
    t = t - jnp.max(t, axis=1, keepdims=True)
    e = jnp.exp(t)
    p = e / jnp.sum(e, axis=1, keepdims=True)
    t2 = jnp.dot(p, wo2_ref[...], preferred_element_type=jnp.float32)
    t2 = t2 + bo2_ref[...]
    t2 = t2 - jnp.max(t2, axis=1, keepdims=True)
    e2 = jnp.exp(t2)
    o_ref[...] = e2 / jnp.sum(e2, axis=1, keepdims=True)


def _final(hw, hg3, bgv, wo1, bo1, wo2, bo2):
    n, wdim = hw.shape
    hdim = bgv.shape[1]
    odim = wo2.shape[1]
    return pl.pallas_call(
        _final_body,
        grid=(n // _BN,),
        in_specs=[
            pl.BlockSpec((_BN, wdim), lambda i: (i, 0)),
            pl.BlockSpec((_K, _BN, wdim), lambda i: (0, i, 0)),
            pl.BlockSpec((1, hdim), lambda i: (0, 0)),
            pl.BlockSpec((hdim, hdim), lambda i: (0, 0)),
            pl.BlockSpec((1, hdim), lambda i: (0, 0)),
            pl.BlockSpec((hdim, odim), lambda i: (0, 0)),
            pl.BlockSpec((1, odim), lambda i: (0, 0)),
        ],
        out_specs=pl.BlockSpec((_BN, odim), lambda i: (i, 0)),
        out_shape=jax.ShapeDtypeStruct((n, odim), jnp.float32),
    )(hw, hg3, bgv, wo1, bo1, wo2, bo2)


def kernel(x, edge_index, W1, b1, W2, b2, Wg, bg, Wo1, bo1, Wo2, bo2):
    n, d = x.shape
    hdim = W1.shape[1]

    w1a = W1[:d]
    w1b = W1[d:]
    wcat = jnp.concatenate([w1a - w1b, w1b], axis=1)

    ab = _ab(x, wcat)
    a = ab[:, :hdim]

    idx = _knn(x)
    idx3d = idx.T.reshape(_NW, -1, _CH)

    bg3 = _sc_gather(ab, idx3d).reshape(_K, n, 2 * hdim)
    hw = _message(a, b1.reshape(1, -1), bg3, W2, b2.reshape(1, -1), Wg)
    hg3 = _sc_gather(hw, idx3d).reshape(_K, n, 2 * hdim)
    return _final(hw, hg3, bg.reshape(1, -1), Wo1, bo1.reshape(1, -1),
                  Wo2, bo2.reshape(1, -1))


# speedup vs baseline: 1.8730x; 1.8730x over previous
import functools

import jax
import jax.numpy as jnp
from jax import lax
from jax.experimental import pallas as pl
from jax.experimental.pallas import tpu as pltpu
from jax.experimental.pallas import tpu_sc as plsc

_K = 16
_BQ = 200
_BN = 1000
_NW = 32
_CH = 40
_F = 5


def _knn_body(xq_ref, xt_ref, idx_ref):
    xq = xq_ref[...]
    xt = xt_ref[...]
    n = xt.shape[1]
    qsq = jnp.sum(xq * xq, axis=1, keepdims=True)
    ksq = jnp.sum(xt * xt, axis=0, keepdims=True)
    mm = jnp.dot(xq, xt, preferred_element_type=jnp.float32)
    neg = -(qsq - 2.0 * mm + ksq)
    cols = lax.broadcasted_iota(jnp.int32, neg.shape, 1)
    outs = []
    for _ in range(_K):
        mx = jnp.max(neg, axis=1, keepdims=True)
        cand = jnp.where(neg == mx, cols, n)
        amin = jnp.min(cand, axis=1, keepdims=True)
        outs.append(amin)
        neg = jnp.where(cols == amin, -jnp.inf, neg)
    idx_ref[...] = jnp.concatenate(outs, axis=1)


def _knn(x):
    n, d = x.shape
    return pl.pallas_call(
        _knn_body,
        grid=(n // _BQ,),
        in_specs=[
            pl.BlockSpec((_BQ, d), lambda i: (i, 0)),
            pl.BlockSpec((d, n), lambda i: (0, 0)),
        ],
        out_specs=pl.BlockSpec((_BQ, _K), lambda i: (i, 0)),
        out_shape=jax.ShapeDtypeStruct((n, _K), jnp.int32),
    )(x, x.T)


def _ab_body(x_ref, w_ref, o_ref):
    o_ref[...] = jnp.dot(x_ref[...], w_ref[...],
                         preferred_element_type=jnp.float32)


def _ab(x, wcat):
    n = x.shape[0]
    return pl.pallas_call(
        _ab_body,
        out_shape=jax.ShapeDtypeStruct((n, wcat.shape[1]), jnp.float32),
    )(x, wcat)


def _sc_gather(table, idx3d):
    nch = idx3d.shape[1]
    rounds = nch // _F
    h = table.shape[1]
    mesh = plsc.VectorSubcoreMesh(core_axis_name="c", subcore_axis_name="s")

    @functools.partial(
        pl.kernel,
        mesh=mesh,
        out_type=jax.ShapeDtypeStruct((_NW * nch, _CH, h), jnp.float32),
        scratch_types=[
            pltpu.VMEM((nch, _CH), jnp.int32),
            pltpu.VMEM((_F, _CH, h), jnp.float32),
            pltpu.SemaphoreType.DMA,
        ],
    )
    def k(table_hbm, idx_hbm, out_hbm, idx_v, buf, sem):
        c = lax.axis_index("c")
        s = lax.axis_index("s")
        wid = s * 2 + c
        pltpu.sync_copy(idx_hbm.at[wid], idx_v)

        def round_body(r, carry):
            cps = [
                pltpu.async_copy(table_hbm.at[idx_v.at[r * _F + j]],
                                 buf.at[j], sem)
                for j in range(_F)
            ]
            for cp in cps:
                cp.wait()
            pltpu.sync_copy(buf, out_hbm.at[pl.ds(wid * nch + r * _F, _F)])
            return carry

        lax.fori_loop(0, rounds, round_body, 0)

    return k(table, idx3d)


def _msg_body(a_ref, b1_ref, bg3_ref, w2_ref, b2_ref, wg_ref, o_ref):
    a = a_ref[...] + b1_ref[...]
    hdim = a.shape[1]
    w2 = w2_ref[...]
    b2 = b2_ref[...]
    hacc = jnp.full(a.shape, -jnp.inf, jnp.float32)
    for k in range(_K):
        t = jnp.dot(jnp.maximum(a + bg3_ref[k][:, hdim:], 0.0), w2,
                    preferred_element_type=jnp.float32) + b2
        mx = jnp.max(t, axis=1, keepdims=True)
        lse = jnp.log(jnp.sum(jnp.exp(t - mx), axis=1, keepdims=True)) + mx
        hacc = jnp.maximum(hacc, t - lse)
    hw = jnp.dot(hacc, wg_ref[...],
                 preferred_element_type=jnp.float32) * (1.0 / 17.0)
    o_ref[...] = jnp.concatenate([hw, jnp.zeros_like(hw)], axis=1)


def _message(a, b1, bg3, w2, b2, wg):
    n, hdim = a.shape
    return pl.pallas_call(
        _msg_body,
        grid=(n // _BN,),
        in_specs=[
            pl.BlockSpec((_BN, hdim), lambda i: (i, 0)),
            pl.BlockSpec((1, hdim), lambda i: (0, 0)),
            pl.BlockSpec((_K, _BN, 2 * hdim), lambda i: (0, i, 0)),
            pl.BlockSpec((hdim, hdim), lambda i: (0, 0)),
            pl.BlockSpec((1, hdim), lambda i: (0, 0)),
            pl.BlockSpec((hdim, hdim), lambda i: (0, 0)),
        ],
        out_specs=pl.BlockSpec((_BN, 2 * hdim), lambda i: (i, 0)),
        out_shape=jax.ShapeDtypeStruct((n, 2 * hdim), jnp.float32),
    )(a, b1, bg3, w2, b2, wg)


def _final_body(hw_ref, hg3_ref, bgv_ref, wo1_ref, bo1_ref, wo2_ref,
                bo2_ref, o_ref):
    hdim = bgv_ref.shape[1]
    g = hw_ref[:, :hdim] + bgv_ref[...]
    for k in range(_K):
        g = g + hg3_ref[k][:, :hdim]
    t = jnp.dot(g, wo1_ref[...], preferred_element_type=jnp.float32)
    t = t + bo1_ref[...]
    t = t - jnp.max(t, axis=1, keepdims=True)
    e = jnp.exp(t)
    p = e / jnp.sum(e, axis=1, keepdims=True)
    t2 = jnp.dot(p, wo2_ref[...], preferred_element_type=jnp.float32)
    t2 = t2 + bo2_ref[...]
    t2 = t2 - jnp.max(t2, axis=1, keepdims=True)
    e2 = jnp.exp(t2)
    o_ref[...] = e2 / jnp.sum(e2, axis=1, keepdims=True)


def _final(hw, hg3, bgv, wo1, bo1, wo2, bo2):
    n, wdim = hw.shape
    hdim = bgv.shape[1]
    odim = wo2.shape[1]
    return pl.pallas_call(
        _final_body,
        grid=(n // _BN,),
        in_specs=[
            pl.BlockSpec((_BN, wdim), lambda i: (i, 0)),
            pl.BlockSpec((_K, _BN, wdim), lambda i: (0, i, 0)),
            pl.BlockSpec((1, hdim), lambda i: (0, 0)),
            pl.BlockSpec((hdim, hdim), lambda i: (0, 0)),
            pl.BlockSpec((1, hdim), lambda i: (0, 0)),
            pl.BlockSpec((hdim, odim), lambda i: (0, 0)),
            pl.BlockSpec((1, odim), lambda i: (0, 0)),
        ],
        out_specs=pl.BlockSpec((_BN, odim), lambda i: (i, 0)),
        out_shape=jax.ShapeDtypeStruct((n, odim), jnp.float32),
    )(hw, hg3, bgv, wo1, bo1, wo2, bo2)


def kernel(x, edge_index, W1, b1, W2, b2, Wg, bg, Wo1, bo1, Wo2, bo2):
    n, d = x.shape
    hdim = W1.shape[1]

    w1a = W1[:d]
    w1b = W1[d:]
    wcat = jnp.concatenate([w1a - w1b, w1b], axis=1)

    ab = _ab(x, wcat)
    a = ab[:, :hdim]

    idx = _knn(x)
    idx3d = idx.T.reshape(_NW, -1, _CH)

    bg3 = _sc_gather(ab, idx3d).reshape(_K, n, 2 * hdim)
    hw = _message(a, b1.reshape(1, -1), bg3, W2, b2.reshape(1, -1), Wg)
    hg3 = _sc_gather(hw, idx3d).reshape(_K, n, 2 * hdim)
    return _final(hw, hg3, bg.reshape(1, -1), Wo1, bo1.reshape(1, -1),
                  Wo2, bo2.reshape(1, -1))
